# baseline (speedup 1.0000x reference)
import functools

import jax
import jax.numpy as jnp
from jax import lax
from jax.experimental import pallas as pl
from jax.experimental.pallas import tpu as pltpu
from jax.experimental.pallas import tpu_sc as plsc

R = 16
N = 4096
L = 16
NCHUNK = N // L


def _sort_row_body(pos_hbm, val_hbm, outv_hbm, outk_hbm,
                   keys_v, vals_v, hist_v, hist2_v, rank_v, occm_v,
                   outk_v, outv_v, sem_k, sem_v, sem_o):
  c = lax.axis_index("c")
  s = lax.axis_index("s")
  wid = s * 2 + c

  @pl.when(wid < R)
  def _():
    row = wid
    h_k = pltpu.make_async_copy(pos_hbm.at[row], keys_v, sem_k)
    h_k.start()
    h_v = pltpu.make_async_copy(val_hbm.at[row], vals_v, sem_v)
    h_v.start()

    cal = plsc.scan_count(lax.iota(jnp.int32, L))[0]

    @plsc.parallel_loop(0, N, step=L, unroll=16)
    def zero_body(i):
      hist_v[pl.ds(i, L)] = jnp.zeros((L,), jnp.int32)
      hist2_v[pl.ds(i, L)] = jnp.zeros((L,), jnp.int32)

    h_k.wait()

    U1 = 8

    @plsc.parallel_loop(0, N, step=L, unroll=U1)
    def scan_only_body(i):
      d = keys_v[pl.ds(i, L)]
      cnt, lastm = plsc.scan_count(d)
      occ = cnt - cal
      rank_v[pl.ds(i, L)] = occ
      occm_v[pl.ds(i, L)] = jnp.where(lastm, occ + 1, 0)

    H = N // 2
    def hist_body(i, carry):
      for u in range(U1):
        off = (i * U1 + u) * L
        d0 = keys_v[pl.ds(off, L)]
        om0 = occm_v[pl.ds(off, L)]
        prev0 = plsc.load_gather(hist_v, [d0])
        rank_v[pl.ds(off, L)] = rank_v[pl.ds(off, L)] + prev0
        plsc.addupdate_scatter(hist_v, [d0], om0, mask=om0 > 0)
        d1 = keys_v[pl.ds(off + H, L)]
        om1 = occm_v[pl.ds(off + H, L)]
        prev1 = plsc.load_gather(hist2_v, [d1])
        rank_v[pl.ds(off + H, L)] = rank_v[pl.ds(off + H, L)] + prev1
        plsc.addupdate_scatter(hist2_v, [d1], om1, mask=om1 > 0)
      return carry

    lax.fori_loop(0, (NCHUNK // 2) // U1, hist_body, jnp.int32(0))

    @plsc.parallel_loop(H, N, step=L, unroll=U1)
    def fixup_body(off):
      d = keys_v[pl.ds(off, L)]
      prev = plsc.load_gather(hist_v, [d])
      rank_v[pl.ds(off, L)] = rank_v[pl.ds(off, L)] + prev

    @plsc.parallel_loop(0, N, step=L, unroll=8, carry=jnp.int32(0))
    def scan_body(i, carry):
      h = hist_v[pl.ds(i, L)] + hist2_v[pl.ds(i, L)]
      incl = plsc.cumsum(h)
      hist_v[pl.ds(i, L)] = incl - h + carry
      return carry + incl[L - 1]

    h_v.wait()

    @plsc.parallel_loop(0, N, step=L, unroll=8)
    def perm_body(i):
      d = keys_v[pl.ds(i, L)]
      v = vals_v[pl.ds(i, L)]
      r = rank_v[pl.ds(i, L)]
      base = plsc.load_gather(hist_v, [d])
      pos = base + r
      plsc.store_scatter(outv_v, [pos], v)
      plsc.store_scatter(outk_v, [pos], d)

    h_ov = pltpu.make_async_copy(outv_v, outv_hbm.at[row], sem_o)
    h_ov.start()
    pltpu.sync_copy(outk_v, outk_hbm.at[row])
    h_ov.wait()


@jax.jit
def kernel(indices, position_indices):
  mesh = plsc.VectorSubcoreMesh(core_axis_name="c", subcore_axis_name="s")
  run = pl.kernel(
      _sort_row_body,
      out_type=(
          jax.ShapeDtypeStruct((R, N), jnp.float32),
          jax.ShapeDtypeStruct((R, N), jnp.int32),
      ),
      mesh=mesh,
      compiler_params=pltpu.CompilerParams(needs_layout_passes=False),
      scratch_types=[
          pltpu.VMEM((N,), jnp.int32),
          pltpu.VMEM((N,), jnp.float32),
          pltpu.VMEM((N,), jnp.int32),
          pltpu.VMEM((N,), jnp.int32),
          pltpu.VMEM((N,), jnp.int32),
          pltpu.VMEM((N,), jnp.int32),
          pltpu.VMEM((N,), jnp.int32),
          pltpu.VMEM((N,), jnp.float32),
          pltpu.SemaphoreType.DMA,
          pltpu.SemaphoreType.DMA,
          pltpu.SemaphoreType.DMA,
      ],
  )
  sorted_vals, sorted_keys = run(position_indices, indices)
  return sorted_vals, sorted_keys


# speedup vs baseline: 1.0133x; 1.0133x over previous
import functools

import jax
import jax.numpy as jnp
from jax import lax
from jax.experimental import pallas as pl
from jax.experimental.pallas import tpu as pltpu
from jax.experimental.pallas import tpu_sc as plsc

R = 16
N = 4096
L = 16
NCHUNK = N // L


def _sort_row_body(pos_hbm, val_hbm, outv_hbm, outk_hbm,
                   keys_v, vals_v, hist_v, rank_v, occm_v, outk_v, outv_v,
                   sem_k, sem_v, sem_o):
  c = lax.axis_index("c")
  s = lax.axis_index("s")
  wid = s * 2 + c

  @pl.when(wid < R)
  def _():
    row = wid
    h_k = pltpu.make_async_copy(pos_hbm.at[row], keys_v, sem_k)
    h_k.start()
    h_v = pltpu.make_async_copy(val_hbm.at[row], vals_v, sem_v)
    h_v.start()

    cal = plsc.scan_count(lax.iota(jnp.int32, L))[0]

    @plsc.parallel_loop(0, N, step=L, unroll=16)
    def zero_body(i):
      hist_v[pl.ds(i, L)] = jnp.zeros((L,), jnp.int32)

    h_k.wait()

    U1 = 16

    @plsc.parallel_loop(0, N, step=L, unroll=U1)
    def scan_only_body(i):
      d = keys_v[pl.ds(i, L)]
      cnt, lastm = plsc.scan_count(d)
      occ = cnt - cal
      rank_v[pl.ds(i, L)] = occ
      occm_v[pl.ds(i, L)] = jnp.where(lastm, occ + 1, 0)

    def hist_body(i, carry):
      for u in range(U1):
        off = (i * U1 + u) * L
        d = keys_v[pl.ds(off, L)]
        om = occm_v[pl.ds(off, L)]
        prev = plsc.load_gather(hist_v, [d])
        rank_v[pl.ds(off, L)] = rank_v[pl.ds(off, L)] + prev
        plsc.addupdate_scatter(hist_v, [d], om, mask=om > 0)
      return carry

    lax.fori_loop(0, NCHUNK // U1, hist_body, jnp.int32(0))

    @plsc.parallel_loop(0, N, step=L, unroll=16, carry=jnp.int32(0))
    def scan_body(i, carry):
      h = hist_v[pl.ds(i, L)]
      incl = plsc.cumsum(h)
      hist_v[pl.ds(i, L)] = incl - h + carry
      return carry + incl[L - 1]

    h_v.wait()

    @plsc.parallel_loop(0, N, step=L, unroll=16)
    def perm_body(i):
      d = keys_v[pl.ds(i, L)]
      v = vals_v[pl.ds(i, L)]
      r = rank_v[pl.ds(i, L)]
      base = plsc.load_gather(hist_v, [d])
      pos = base + r
      plsc.store_scatter(outv_v, [pos], v)
      plsc.store_scatter(outk_v, [pos], d)

    h_ov = pltpu.make_async_copy(outv_v, outv_hbm.at[row], sem_o)
    h_ov.start()
    pltpu.sync_copy(outk_v, outk_hbm.at[row])
    h_ov.wait()


@jax.jit
def kernel(indices, position_indices):
  mesh = plsc.VectorSubcoreMesh(core_axis_name="c", subcore_axis_name="s")
  run = pl.kernel(
      _sort_row_body,
      out_type=(
          jax.ShapeDtypeStruct((R, N), jnp.float32),
          jax.ShapeDtypeStruct((R, N), jnp.int32),
      ),
      mesh=mesh,
      compiler_params=pltpu.CompilerParams(needs_layout_passes=False),
      scratch_types=[
          pltpu.VMEM((N,), jnp.int32),
          pltpu.VMEM((N,), jnp.float32),
          pltpu.VMEM((N,), jnp.int32),
          pltpu.VMEM((N,), jnp.int32),
          pltpu.VMEM((N,), jnp.int32),
          pltpu.VMEM((N,), jnp.int32),
          pltpu.VMEM((N,), jnp.float32),
          pltpu.SemaphoreType.DMA,
          pltpu.SemaphoreType.DMA,
          pltpu.SemaphoreType.DMA,
      ],
  )
  sorted_vals, sorted_keys = run(position_indices, indices)
  return sorted_vals, sorted_keys
